# baseline (speedup 1.0000x reference)
import functools
from math import sqrt

import jax
import jax.numpy as jnp
from jax import lax
from jax.experimental import pallas as pl
from jax.experimental.pallas import tpu as pltpu
from jax.experimental.pallas import tpu_sc as plsc

D_MODEL = 64
SCALE = float(sqrt(D_MODEL))
LANES = 16

NUM_CORES = 2
NUM_SUBCORES = 16
NUM_WORKERS = NUM_CORES * NUM_SUBCORES


@functools.lru_cache(maxsize=None)
def _make_lookup(B: int, D: int, C: int):
    assert B % (8 * NUM_WORKERS) == 0
    b_per_w = B // NUM_WORKERS
    assert b_per_w % C == 0
    n_chunks = b_per_w // C
    mesh = plsc.VectorSubcoreMesh(core_axis_name="c", subcore_axis_name="s")

    @functools.partial(
        pl.kernel,
        mesh=mesh,
        out_type=jax.ShapeDtypeStruct((B, D), jnp.float32),
        scratch_types=[
            pltpu.VMEM((b_per_w,), jnp.int32),
            pltpu.VMEM((C, D), jnp.float32),
            pltpu.SemaphoreType.DMA,
        ],
        compiler_params=pltpu.CompilerParams(use_tc_tiling_on_sc=False),
    )
    def lookup(idx_hbm, table_hbm, out_hbm, idx_v, rows_v, sem):
        wid = lax.axis_index("s") * NUM_CORES + lax.axis_index("c")
        base = wid * b_per_w
        pltpu.sync_copy(idx_hbm.at[pl.ds(base, b_per_w)], idx_v)

        def chunk_body(ci, carry):
            off = ci * C
            pltpu.async_copy(
                table_hbm.at[idx_v.at[pl.ds(off, C)]], rows_v, sem
            ).wait()

            def row_body(r, c2):
                for j in range(D // LANES):
                    sl = pl.ds(j * LANES, LANES)
                    rows_v[r, sl] = rows_v[r, sl] * SCALE
                return c2

            lax.fori_loop(0, C, row_body, 0, unroll=4)
            pltpu.sync_copy(rows_v, out_hbm.at[pl.ds(base + off, C)])
            return carry

        lax.fori_loop(0, n_chunks, chunk_body, 0)

    return lookup


def kernel(x, table):
    B = x.shape[0] * x.shape[1]
    idx = x.reshape(-1).astype(jnp.int32)
    out = _make_lookup(B, D_MODEL, 1024)(idx, table)
    return out.reshape(x.shape[0], x.shape[1], D_MODEL)


# speedup vs baseline: 1.0511x; 1.0511x over previous
import functools
from math import sqrt

import jax
import jax.numpy as jnp
from jax import lax
from jax.experimental import pallas as pl
from jax.experimental.pallas import tpu as pltpu
from jax.experimental.pallas import tpu_sc as plsc

D_MODEL = 64
SCALE = float(sqrt(D_MODEL))
LANES = 16

NUM_CORES = 2
NUM_SUBCORES = 16
NUM_WORKERS = NUM_CORES * NUM_SUBCORES

NBUF = 4
CHUNK = 256


@functools.lru_cache(maxsize=None)
def _make_lookup(B: int, D: int):
    C = CHUNK
    assert B % (8 * NUM_WORKERS) == 0
    b_per_w = B // NUM_WORKERS
    assert b_per_w % (C * NBUF) == 0
    n_chunks = b_per_w // C
    n_outer = n_chunks // NBUF
    mesh = plsc.VectorSubcoreMesh(core_axis_name="c", subcore_axis_name="s")

    @functools.partial(
        pl.kernel,
        mesh=mesh,
        out_type=jax.ShapeDtypeStruct((B, D), jnp.float32),
        scratch_types=(
            [pltpu.VMEM((b_per_w,), jnp.int32)]
            + [pltpu.VMEM((C, D), jnp.float32) for _ in range(NBUF)]
            + [pltpu.SemaphoreType.DMA for _ in range(NBUF)]
        ),
        compiler_params=pltpu.CompilerParams(use_tc_tiling_on_sc=False),
    )
    def lookup(idx_hbm, table_hbm, out_hbm, idx_v, b0, b1, b2, b3, s0, s1, s2, s3):
        bufs = [b0, b1, b2, b3]
        sems = [s0, s1, s2, s3]
        wid = lax.axis_index("s") * NUM_CORES + lax.axis_index("c")
        base = wid * b_per_w
        pltpu.sync_copy(idx_hbm.at[pl.ds(base, b_per_w)], idx_v)

        def start_gather(g, b):
            pltpu.async_copy(
                table_hbm.at[idx_v.at[pl.ds(g * C, C)]], bufs[b], sems[b]
            )

        def wait_gather(g, b):
            pltpu.make_async_copy(
                table_hbm.at[idx_v.at[pl.ds(g * C, C)]], bufs[b], sems[b]
            ).wait()

        def start_store(g, b):
            pltpu.async_copy(bufs[b], out_hbm.at[pl.ds(base + g * C, C)], sems[b])

        def wait_store(b):
            pltpu.make_async_copy(
                bufs[b], out_hbm.at[pl.ds(base, C)], sems[b]
            ).wait()

        def scale_buf(b):
            buf = bufs[b]

            @plsc.parallel_loop(0, C, 1, unroll=8)
            def _(r):
                for j in range(D // LANES):
                    sl = pl.ds(j * LANES, LANES)
                    buf[r, sl] = buf[r, sl] * SCALE

        for b in range(NBUF - 1):
            start_gather(b, b)

        def outer(o, carry):
            for b in range(NBUF):
                g = o * NBUF + b
                wait_gather(g, b)
                scale_buf(b)
                start_store(g, b)
                bp = (b - 1) % NBUF
                g_next = g + NBUF - 1

                @pl.when(g_next < n_chunks)
                def _():
                    @pl.when(g > 0)
                    def _():
                        wait_store(bp)

                    start_gather(g_next, bp)

            return carry

        lax.fori_loop(0, n_outer, outer, 0)

        for b in range(NBUF):
            wait_store(b)

    return lookup


def kernel(x, table):
    B = x.shape[0] * x.shape[1]
    idx = x.reshape(-1).astype(jnp.int32)
    out = _make_lookup(B, D_MODEL)(idx, table)
    return out.reshape(x.shape[0], x.shape[1], D_MODEL)
